# baseline (speedup 1.0000x reference)
import functools

import jax
import jax.numpy as jnp
from jax import lax
from jax.experimental import pallas as pl
from jax.experimental.pallas import tpu as pltpu
from jax.experimental.pallas import tpu_sc as plsc

NC = 2
NS = 16
NW = NC * NS
G = 128


def kernel(token_ids, weights):
    B = token_ids.shape[0] * token_ids.shape[1]
    D = weights.shape[1]
    chunks = B // (NW * G)
    assert B == NW * chunks * G
    idx = token_ids.reshape(NW, chunks, G).astype(jnp.int32)

    @functools.partial(
        pl.kernel,
        mesh=plsc.VectorSubcoreMesh(core_axis_name="c", subcore_axis_name="s"),
        out_type=jax.ShapeDtypeStruct((B, D), jnp.float32),
        compiler_params=pltpu.CompilerParams(use_tc_tiling_on_sc=False),
        scratch_types=[
            pltpu.VMEM((chunks, G), jnp.int32),
            pltpu.VMEM((G, D), jnp.float32),
            pltpu.SemaphoreType.DMA,
        ],
    )
    def gather_k(idx_hbm, table_hbm, out_hbm, idx_v, rows_v, sem):
        wid = lax.axis_index("s") * NC + lax.axis_index("c")
        base = wid * (chunks * G)
        pltpu.sync_copy(idx_hbm.at[wid], idx_v)

        def body(j, carry):
            pltpu.async_copy(table_hbm.at[idx_v.at[j]], rows_v, sem).wait()
            pltpu.sync_copy(rows_v, out_hbm.at[pl.ds(base + j * G, G)])
            return carry

        lax.fori_loop(0, chunks, body, 0)

    out = gather_k(idx, weights)
    return out.reshape(token_ids.shape + (D,))


# speedup vs baseline: 1.1157x; 1.1157x over previous
import functools

import jax
import jax.numpy as jnp
from jax import lax
from jax.experimental import pallas as pl
from jax.experimental.pallas import tpu as pltpu
from jax.experimental.pallas import tpu_sc as plsc

NC = 2
NS = 16
NW = NC * NS
G = 128
S = 8
P = 5


def kernel(token_ids, weights):
    B = token_ids.shape[0] * token_ids.shape[1]
    D = weights.shape[1]
    chunks = B // (NW * G)
    assert B == NW * chunks * G and chunks % S == 0 and chunks >= 2 * S
    idx = token_ids.reshape(NW, chunks, G).astype(jnp.int32)

    @functools.partial(
        pl.kernel,
        mesh=plsc.VectorSubcoreMesh(core_axis_name="c", subcore_axis_name="s"),
        out_type=jax.ShapeDtypeStruct((B, D), jnp.float32),
        compiler_params=pltpu.CompilerParams(use_tc_tiling_on_sc=False),
        scratch_types=[
            pltpu.VMEM((chunks, G), jnp.int32),
            pltpu.VMEM((S, G, D), jnp.float32),
            [pltpu.SemaphoreType.DMA] * S,
            [pltpu.SemaphoreType.DMA] * S,
        ],
    )
    def gather_k(idx_hbm, table_hbm, out_hbm, idx_v, rows_v, gsem, psem):
        wid = lax.axis_index("s") * NC + lax.axis_index("c")
        base = wid * (chunks * G)
        pltpu.sync_copy(idx_hbm.at[wid], idx_v)

        def fire_gather(m, slot):
            pltpu.async_copy(table_hbm.at[idx_v.at[m]], rows_v.at[slot],
                             gsem[slot])

        def wait_gather(slot):
            pltpu.make_async_copy(table_hbm.at[idx_v.at[0]], rows_v.at[slot],
                                  gsem[slot]).wait()

        def fire_put(j, slot):
            pltpu.async_copy(rows_v.at[slot], out_hbm.at[pl.ds(base + j * G, G)],
                             psem[slot])

        def wait_put(slot):
            pltpu.make_async_copy(rows_v.at[slot], out_hbm.at[pl.ds(base, G)],
                                  psem[slot]).wait()

        def step(j, b, first_round):
            mb = (b + P) % S
            if not (first_round and b < S - P):
                wait_put(mb)
            fire_gather(j + P, mb)
            wait_gather(b)
            fire_put(j, b)

        for m in range(P):
            fire_gather(m, m)

        for b in range(S):
            step(b, b, True)

        def outer(g, carry):
            for b in range(S):
                step(g * S + b, b, False)
            return carry

        lax.fori_loop(1, chunks // S - 1, outer, 0)

        for b in range(S):
            j = (chunks - S) + b
            if j + P < chunks:
                mb = (b + P) % S
                wait_put(mb)
                fire_gather(j + P, mb)
            wait_gather(b)
            fire_put(j, b)
        for b in range(S):
            wait_put(b)

    out = gather_k(idx, weights)
    return out.reshape(token_ids.shape + (D,))
